# Initial kernel scaffold; baseline (speedup 1.0000x reference)
import jax
import jax.numpy as jnp
from jax.experimental import pallas as pl


def kernel(x, N1, N2, gcn1_W, gcn1_b, gcn2_W, gcn2_b, W_ih, W_hh, b_ih, b_hh, fc_W, fc_b):
    raise NotImplementedError("write your pallas kernel here")


# speedup vs baseline: 884.7404x; 884.7404x over previous
import jax
import jax.numpy as jnp
from jax.experimental import pallas as pl
from jax.experimental.pallas import tpu as pltpu

_B, _N, _S, _F, _H = 4, 100, 24, 6, 32
_G = _B * _S
_NT = _G * _N
_BN = _B * _N
_DEG_INV = 1.0 / float(_N + 1)


def _fused_body(x_ref, w1_ref, b1_ref, w2_ref, b2_ref,
                wih_ref, whh_ref, bg_ref, fcw_ref, fcb_ref, out_ref,
                seq_ref):
    t = jnp.dot(x_ref[...], w1_ref[...], preferred_element_type=jnp.float32)
    t3 = t.reshape(_G, _N, _H)
    s = jnp.sum(t3, axis=1, keepdims=True)
    h = jnp.maximum((t3 + s) * _DEG_INV + b1_ref[...].reshape(1, 1, _H), 0.0)

    t = jnp.dot(h.reshape(_NT, _H), w2_ref[...],
                preferred_element_type=jnp.float32)
    t3 = t.reshape(_G, _N, _H)
    s = jnp.sum(t3, axis=1, keepdims=True)
    h = jnp.maximum((t3 + s) * _DEG_INV + b2_ref[...].reshape(1, 1, _H), 0.0)

    gx = jnp.dot(h.reshape(_NT, _H), wih_ref[...],
                 preferred_element_type=jnp.float32) + bg_ref[...]
    gx4 = gx.reshape(_B, _S, _N, 4 * _H)
    for b in range(_B):
        seq_ref[:, b * _N:(b + 1) * _N, :] = gx4[b]

    def step(tstep, carry):
        hs, cs = carry
        g = seq_ref[tstep]
        g = g + jnp.dot(hs, whh_ref[...], preferred_element_type=jnp.float32)
        i = jax.nn.sigmoid(g[:, 0:_H])
        f = jax.nn.sigmoid(g[:, _H:2 * _H])
        cand = jnp.tanh(g[:, 2 * _H:3 * _H])
        o = jax.nn.sigmoid(g[:, 3 * _H:4 * _H])
        cs = f * cs + i * cand
        hs = o * jnp.tanh(cs)
        return (hs, cs)

    zeros = jnp.zeros((_BN, _H), jnp.float32)
    hs, cs = jax.lax.fori_loop(0, _S, step, (zeros, zeros))

    out_ref[...] = (jnp.sum(hs * fcw_ref[...], axis=1, keepdims=True)
                    + fcb_ref[0, 0])


@jax.jit
def _run(x, gcn1_W, gcn1_b, gcn2_W, gcn2_b,
         W_ih, W_hh, b_ih, b_hh, fc_W, fc_b):
    xt = jnp.transpose(x, (0, 2, 1, 3)).reshape(_NT, _F)
    bg = (b_ih + b_hh).reshape(1, 4 * _H)
    out = pl.pallas_call(
        _fused_body,
        out_shape=jax.ShapeDtypeStruct((_BN, 1), jnp.float32),
        scratch_shapes=[pltpu.VMEM((_S, _BN, 4 * _H), jnp.float32)],
    )(xt, gcn1_W, gcn1_b.reshape(1, _H), gcn2_W, gcn2_b.reshape(1, _H),
      W_ih.T, W_hh.T, bg, fc_W, fc_b.reshape(1, 1))
    return out.reshape(_B, _N, 1)


def kernel(x, N1, N2, gcn1_W, gcn1_b, gcn2_W, gcn2_b,
           W_ih, W_hh, b_ih, b_hh, fc_W, fc_b):
    del N1, N2
    return _run(x, gcn1_W, gcn1_b, gcn2_W, gcn2_b,
                W_ih, W_hh, b_ih, b_hh, fc_W, fc_b)


# speedup vs baseline: 1069.6745x; 1.2090x over previous
import jax
import jax.numpy as jnp
from jax.experimental import pallas as pl

_B, _N, _S, _F, _H = 4, 100, 24, 6, 32
_BN = _B * _N
_DEG_INV = 1.0 / float(_N + 1)


def _gcn_combine(t, bias):
    s = jnp.sum(t.reshape(_B, _N, _S * _H), axis=1, keepdims=True)
    h = (t.reshape(_B, _N, _S * _H) + s) * _DEG_INV + bias.reshape(1, 1, -1)
    return jnp.maximum(h, 0.0).reshape(_BN, _S * _H)


def _fused_body(x_ref, w1k_ref, b1_ref, w2_ref, b2_ref,
                wih_ref, whh_ref, bg_ref, fcw_ref, fcb_ref, out_ref):
    t = jnp.dot(x_ref[...], w1k_ref[...], preferred_element_type=jnp.float32)
    h = _gcn_combine(t, b1_ref[...])

    w2 = w2_ref[...]
    t = jnp.concatenate(
        [jnp.dot(h[:, s * _H:(s + 1) * _H], w2,
                 preferred_element_type=jnp.float32) for s in range(_S)],
        axis=1)
    h = _gcn_combine(t, b2_ref[...])

    wih = wih_ref[...]
    whh = whh_ref[...]
    bg = bg_ref[...]
    hs = jnp.zeros((_BN, _H), jnp.float32)
    cs = jnp.zeros((_BN, _H), jnp.float32)
    for s in range(_S):
        g = (jnp.dot(h[:, s * _H:(s + 1) * _H], wih,
                     preferred_element_type=jnp.float32)
             + jnp.dot(hs, whh, preferred_element_type=jnp.float32) + bg)
        i = jax.nn.sigmoid(g[:, 0:_H])
        f = jax.nn.sigmoid(g[:, _H:2 * _H])
        cand = jnp.tanh(g[:, 2 * _H:3 * _H])
        o = jax.nn.sigmoid(g[:, 3 * _H:4 * _H])
        cs = f * cs + i * cand
        hs = o * jnp.tanh(cs)

    out_ref[...] = (jnp.sum(hs * fcw_ref[...], axis=1, keepdims=True)
                    + fcb_ref[0, 0])


@jax.jit
def _run(x, gcn1_W, gcn1_b, gcn2_W, gcn2_b,
         W_ih, W_hh, b_ih, b_hh, fc_W, fc_b):
    x2 = x.reshape(_BN, _S * _F)
    w1k = jnp.kron(jnp.eye(_S, dtype=x.dtype), gcn1_W)
    bg = (b_ih + b_hh).reshape(1, 4 * _H)
    out = pl.pallas_call(
        _fused_body,
        out_shape=jax.ShapeDtypeStruct((_BN, 1), jnp.float32),
    )(x2, w1k, jnp.tile(gcn1_b, _S).reshape(1, _S * _H),
      gcn2_W, jnp.tile(gcn2_b, _S).reshape(1, _S * _H),
      W_ih.T, W_hh.T, bg, fc_W, fc_b.reshape(1, 1))
    return out.reshape(_B, _N, 1)


def kernel(x, N1, N2, gcn1_W, gcn1_b, gcn2_W, gcn2_b,
           W_ih, W_hh, b_ih, b_hh, fc_W, fc_b):
    del N1, N2
    return _run(x, gcn1_W, gcn1_b, gcn2_W, gcn2_b,
                W_ih, W_hh, b_ih, b_hh, fc_W, fc_b)


# speedup vs baseline: 1157.4837x; 1.0821x over previous
import jax
import jax.numpy as jnp
from jax.experimental import pallas as pl
from jax.experimental.pallas import tpu as pltpu

_B, _N, _S, _F, _H = 4, 100, 24, 6, 32
_BN = _B * _N


def _gcn_combine(t, bias):
    dis = jax.lax.rsqrt(jnp.full((1, 1), float(_N + 1), jnp.float32))
    msg = t.reshape(_B, _N, _S * _H) * (dis * dis)[0, 0]
    s = jnp.sum(msg, axis=1, keepdims=True)
    h = msg + s + bias.reshape(1, 1, -1)
    return jnp.maximum(h, 0.0).reshape(_BN, _S * _H)


def _ifog(w):
    return jnp.concatenate([w[0:2 * _H], w[3 * _H:4 * _H], w[2 * _H:3 * _H]],
                           axis=0)


def _fused_body(x_ref, w1_ref, b1_ref, w2_ref, b2_ref,
                wih_ref, whh_ref, bih_ref, bhh_ref, fcw_ref, fcb_ref,
                out_ref, w1k_ref):
    w1k_ref[...] = jnp.zeros((_S * _F, _S * _H), jnp.float32)
    w1 = w1_ref[...]
    for s in range(_S):
        w1k_ref[s * _F:(s + 1) * _F, s * _H:(s + 1) * _H] = w1

    t = jnp.dot(x_ref[...], w1k_ref[...], preferred_element_type=jnp.float32)
    b1 = jnp.concatenate([b1_ref[...]] * _S, axis=1)
    h = _gcn_combine(t, b1)

    w2 = w2_ref[...]
    t = jnp.concatenate(
        [jnp.dot(h[:, s * _H:(s + 1) * _H], w2,
                 preferred_element_type=jnp.float32) for s in range(_S)],
        axis=1)
    b2 = jnp.concatenate([b2_ref[...]] * _S, axis=1)
    h = _gcn_combine(t, b2)

    _C = 2
    _R = _BN // _C
    wih = jnp.transpose(_ifog(wih_ref[...]))
    whh = jnp.transpose(_ifog(whh_ref[...]))
    bg = jnp.transpose(
        _ifog(jnp.transpose(bih_ref[...] + bhh_ref[...])))
    hs = [jnp.zeros((_R, _H), jnp.float32) for _ in range(_C)]
    cs = [jnp.zeros((_R, _H), jnp.float32) for _ in range(_C)]
    for s in range(_S):
        xs = h[:, s * _H:(s + 1) * _H]
        for c in range(_C):
            g = (jnp.dot(xs[c * _R:(c + 1) * _R], wih,
                         preferred_element_type=jnp.float32)
                 + jnp.dot(hs[c], whh, preferred_element_type=jnp.float32)
                 + bg)
            sig = jax.nn.sigmoid(g[:, 0:3 * _H])
            i = sig[:, 0:_H]
            f = sig[:, _H:2 * _H]
            o = sig[:, 2 * _H:3 * _H]
            cand = jnp.tanh(g[:, 3 * _H:4 * _H])
            cs[c] = f * cs[c] + i * cand
            hs[c] = o * jnp.tanh(cs[c])

    fcw = fcw_ref[...]
    for c in range(_C):
        out_ref[c * _R:(c + 1) * _R, :] = (
            jnp.sum(hs[c] * fcw, axis=1, keepdims=True) + fcb_ref[0, 0])


@jax.jit
def _run(x, gcn1_W, gcn1_b, gcn2_W, gcn2_b,
         W_ih, W_hh, b_ih, b_hh, fc_W, fc_b):
    x2 = x.reshape(_BN, _S * _F)
    out = pl.pallas_call(
        _fused_body,
        out_shape=jax.ShapeDtypeStruct((_BN, 1), jnp.float32),
        scratch_shapes=[pltpu.VMEM((_S * _F, _S * _H), jnp.float32)],
    )(x2, gcn1_W, gcn1_b.reshape(1, _H), gcn2_W, gcn2_b.reshape(1, _H),
      W_ih, W_hh, b_ih.reshape(1, 4 * _H), b_hh.reshape(1, 4 * _H),
      fc_W, fc_b.reshape(1, 1))
    return out.reshape(_B, _N, 1)


def kernel(x, N1, N2, gcn1_W, gcn1_b, gcn2_W, gcn2_b,
           W_ih, W_hh, b_ih, b_hh, fc_W, fc_b):
    del N1, N2
    return _run(x, gcn1_W, gcn1_b, gcn2_W, gcn2_b,
                W_ih, W_hh, b_ih, b_hh, fc_W, fc_b)
